# baseline (speedup 1.0000x reference)
import functools

import jax
import jax.numpy as jnp
from jax import lax
from jax.experimental import pallas as pl
from jax.experimental.pallas import tpu as pltpu
from jax.experimental.pallas import tpu_sc as plsc

D_MODEL = 64
SCALE = 8.0
LANES = 16

NC = 2
NS = 16
NW = NC * NS

B_TOTAL = 1024 * 200
B_PER_W = B_TOTAL // NW
CHUNK = 128
N_CHUNKS = B_PER_W // CHUNK
NBUF = 2

_mesh = plsc.VectorSubcoreMesh(core_axis_name="c", subcore_axis_name="s")


@functools.partial(
    pl.kernel,
    mesh=_mesh,
    compiler_params=pltpu.CompilerParams(use_tc_tiling_on_sc=True),
    out_type=jax.ShapeDtypeStruct((B_TOTAL, D_MODEL), jnp.float32),
    scratch_types=[
        pltpu.VMEM((N_CHUNKS, CHUNK), jnp.int32),
        pltpu.VMEM((N_CHUNKS, CHUNK), jnp.int32),
        pltpu.VMEM((NBUF, CHUNK, 2 * D_MODEL), jnp.float32),
        pltpu.VMEM((NBUF, CHUNK, D_MODEL), jnp.float32),
        pltpu.SemaphoreType.DMA,
        pltpu.SemaphoreType.DMA,
        pltpu.SemaphoreType.DMA,
        pltpu.SemaphoreType.DMA,
    ],
)
def _emb_lookup(xp_hbm, xq_hbm, tpair_hbm, out_hbm, pidx_v, poff_v,
                gbuf, sbuf, gsem0, gsem1, ssem0, ssem1):
    gsems = (gsem0, gsem1)
    ssems = (ssem0, ssem1)
    wid = lax.axis_index("s") * NC + lax.axis_index("c")
    base = wid * B_PER_W

    pltpu.sync_copy(xp_hbm.at[wid], pidx_v)
    pltpu.sync_copy(xq_hbm.at[wid], poff_v)

    def gather_copy(g, slot):
        return pltpu.make_async_copy(
            tpair_hbm.at[pidx_v.at[g]], gbuf.at[slot], gsems[slot])

    def scatter_copy(g, slot):
        return pltpu.make_async_copy(
            sbuf.at[slot], out_hbm.at[pl.ds(base + g * CHUNK, CHUNK)],
            ssems[slot])

    def scale_chunk(g, slot):
        def grp(k, carry):
            offs = poff_v[g, pl.ds(k * LANES, LANES)]
            for l in range(LANES):
                off = offs[l]
                r = k * LANES + l
                for c in range(D_MODEL // LANES):
                    sbuf[slot, r, pl.ds(c * LANES, LANES)] = (
                        gbuf[slot, r, pl.ds(off + c * LANES, LANES)] * SCALE)
            return carry
        lax.fori_loop(0, CHUNK // LANES, grp, 0)

    for b in range(NBUF):
        gather_copy(b, b).start()

    for b in range(NBUF):
        gather_copy(b, b).wait()
        scale_chunk(b, b)
        gather_copy(b + NBUF, b).start()
        scatter_copy(b, b).start()

    def outer(j, carry):
        for b in range(NBUF):
            g = j * NBUF + b
            gather_copy(g, b).wait()
            scatter_copy(g - NBUF, b).wait()
            scale_chunk(g, b)
            gather_copy(g + NBUF, b).start()
            scatter_copy(g, b).start()
        return carry
    lax.fori_loop(1, N_CHUNKS // NBUF - 1, outer, 0)

    for b in range(NBUF):
        g = N_CHUNKS - NBUF + b
        gather_copy(g, b).wait()
        scatter_copy(g - NBUF, b).wait()
        scale_chunk(g, b)
        scatter_copy(g, b).start()

    for b in range(NBUF):
        scatter_copy(N_CHUNKS - NBUF + b, b).wait()


_TBLK = 2048


def _transpose_body(xt_ref, out_ref):
    ident = jnp.eye(D_MODEL, dtype=jnp.float32)
    dn = (((0,), (0,)), ((), ()))
    lo = lax.dot_general(xt_ref[:, : _TBLK // 2], ident, dn,
                         preferred_element_type=jnp.float32)
    hi = lax.dot_general(xt_ref[:, _TBLK // 2 :], ident, dn,
                         preferred_element_type=jnp.float32)
    out_ref[...] = lax.concatenate([lo, hi], 1)


_NBLK = -(-1000000 // _TBLK)

_transpose_table = pl.pallas_call(
    _transpose_body,
    grid=(_NBLK,),
    in_specs=[pl.BlockSpec((D_MODEL, _TBLK), lambda i: (0, i))],
    out_specs=pl.BlockSpec((_TBLK // 2, 2 * D_MODEL), lambda i: (i, 0)),
    out_shape=jax.ShapeDtypeStruct((_NBLK * _TBLK // 2, 2 * D_MODEL),
                                   jnp.float32),
)


def kernel(x, emb_table):
    xi = x.astype(jnp.int32)
    xp = (((xi >> 11) << 10) | (xi & 1023)).reshape(NW, N_CHUNKS, CHUNK)
    xq = (((xi >> 10) & 1) * D_MODEL).reshape(NW, N_CHUNKS, CHUNK)
    tpair = _transpose_table(emb_table.T)
    out = _emb_lookup(xp, xq, tpair)
    return out.reshape(x.shape[0], x.shape[1], D_MODEL)


# speedup vs baseline: 1.4392x; 1.4392x over previous
import functools

import jax
import jax.numpy as jnp
from jax import lax
from jax.experimental import pallas as pl
from jax.experimental.pallas import tpu as pltpu
from jax.experimental.pallas import tpu_sc as plsc

D_MODEL = 64
SCALE = 8.0
LANES = 16

NC = 2
NS = 16
NW = NC * NS

B_TOTAL = 1024 * 200
B_PER_W = B_TOTAL // NW
CHUNK = 128
N_CHUNKS = B_PER_W // CHUNK
NBUF = 2

_mesh = plsc.VectorSubcoreMesh(core_axis_name="c", subcore_axis_name="s")


@functools.partial(
    pl.kernel,
    mesh=_mesh,
    compiler_params=pltpu.CompilerParams(use_tc_tiling_on_sc=True),
    out_type=jax.ShapeDtypeStruct((B_TOTAL, D_MODEL), jnp.float32),
    scratch_types=[
        pltpu.VMEM((N_CHUNKS, CHUNK), jnp.int32),
        pltpu.VMEM((N_CHUNKS, CHUNK), jnp.int32),
        pltpu.VMEM((NBUF, CHUNK, 2 * D_MODEL), jnp.float32),
        pltpu.VMEM((NBUF, CHUNK, D_MODEL), jnp.float32),
        pltpu.SemaphoreType.DMA,
        pltpu.SemaphoreType.DMA,
        pltpu.SemaphoreType.DMA,
        pltpu.SemaphoreType.DMA,
    ],
)
def _emb_lookup(xp_hbm, xq_hbm, tpair_hbm, out_hbm, pidx_v, poff_v,
                gbuf, sbuf, gsem0, gsem1, ssem0, ssem1):
    gsems = (gsem0, gsem1)
    ssems = (ssem0, ssem1)
    wid = lax.axis_index("s") * NC + lax.axis_index("c")
    base = wid * B_PER_W

    pltpu.sync_copy(xp_hbm.at[wid], pidx_v)
    pltpu.sync_copy(xq_hbm.at[wid], poff_v)

    def gather_copy(g, slot):
        return pltpu.make_async_copy(
            tpair_hbm.at[pidx_v.at[g]], gbuf.at[slot], gsems[slot])

    def scatter_copy(g, slot):
        return pltpu.make_async_copy(
            sbuf.at[slot], out_hbm.at[pl.ds(base + g * CHUNK, CHUNK)],
            ssems[slot])

    def scale_chunk(g, slot):
        def grp(k, carry):
            offs = poff_v[g, pl.ds(k * LANES, LANES)]
            for l in range(LANES):
                off = offs[l]
                r = k * LANES + l
                for c in range(D_MODEL // LANES):
                    sbuf[slot, r, pl.ds(c * LANES, LANES)] = (
                        gbuf[slot, r, pl.ds(off + c * LANES, LANES)] * SCALE)
            return carry
        lax.fori_loop(0, CHUNK // LANES, grp, 0)

    for b in range(NBUF):
        gather_copy(b, b).start()

    for b in range(NBUF):
        gather_copy(b, b).wait()
        scale_chunk(b, b)
        gather_copy(b + NBUF, b).start()
        scatter_copy(b, b).start()

    def outer(j, carry):
        for b in range(NBUF):
            g = j * NBUF + b
            gather_copy(g, b).wait()
            scatter_copy(g - NBUF, b).wait()
            scale_chunk(g, b)
            gather_copy(g + NBUF, b).start()
            scatter_copy(g, b).start()
        return carry
    lax.fori_loop(1, N_CHUNKS // NBUF - 1, outer, 0)

    for b in range(NBUF):
        g = N_CHUNKS - NBUF + b
        gather_copy(g, b).wait()
        scatter_copy(g - NBUF, b).wait()
        scale_chunk(g, b)
        scatter_copy(g, b).start()

    for b in range(NBUF):
        scatter_copy(N_CHUNKS - NBUF + b, b).wait()


_TBLK = 8192
_TSH = 13


def _transpose_body(xt_ref, out_ref):
    lo = xt_ref[:, : _TBLK // 2].T
    hi = xt_ref[:, _TBLK // 2 :].T
    out_ref[...] = lax.concatenate([lo, hi], 1)


_NBLK = -(-1000000 // _TBLK)

_transpose_table = pl.pallas_call(
    _transpose_body,
    grid=(_NBLK,),
    in_specs=[pl.BlockSpec((D_MODEL, _TBLK), lambda i: (0, i))],
    out_specs=pl.BlockSpec((_TBLK // 2, 2 * D_MODEL), lambda i: (i, 0)),
    out_shape=jax.ShapeDtypeStruct((_NBLK * _TBLK // 2, 2 * D_MODEL),
                                   jnp.float32),
)


def kernel(x, emb_table):
    xi = x.astype(jnp.int32)
    xp = (((xi >> _TSH) << (_TSH - 1))
          | (xi & (_TBLK // 2 - 1))).reshape(NW, N_CHUNKS, CHUNK)
    xq = (((xi >> (_TSH - 1)) & 1) * D_MODEL).reshape(NW, N_CHUNKS, CHUNK)
    tpair = _transpose_table(emb_table.T)
    out = _emb_lookup(xp, xq, tpair)
    return out.reshape(x.shape[0], x.shape[1], D_MODEL)


# speedup vs baseline: 1.5552x; 1.0806x over previous
import functools

import jax
import jax.numpy as jnp
from jax import lax
from jax.experimental import pallas as pl
from jax.experimental.pallas import tpu as pltpu
from jax.experimental.pallas import tpu_sc as plsc

D_MODEL = 64
SCALE = 8.0
LANES = 16

NC = 2
NS = 16
NW = NC * NS

B_TOTAL = 1024 * 200
B_PER_W = B_TOTAL // NW
CHUNK = 128
N_CHUNKS = B_PER_W // CHUNK
NBUF = 2

_mesh = plsc.VectorSubcoreMesh(core_axis_name="c", subcore_axis_name="s")


@functools.partial(
    pl.kernel,
    mesh=_mesh,
    compiler_params=pltpu.CompilerParams(use_tc_tiling_on_sc=True),
    out_type=jax.ShapeDtypeStruct((B_TOTAL, D_MODEL), jnp.float32),
    scratch_types=[
        pltpu.VMEM((N_CHUNKS, CHUNK), jnp.int32),
        pltpu.VMEM((N_CHUNKS, CHUNK), jnp.int32),
        pltpu.VMEM((NBUF, CHUNK, 2 * D_MODEL), jnp.float32),
        pltpu.VMEM((NBUF, CHUNK, D_MODEL), jnp.float32),
        pltpu.SemaphoreType.DMA,
        pltpu.SemaphoreType.DMA,
        pltpu.SemaphoreType.DMA,
        pltpu.SemaphoreType.DMA,
    ],
)
def _emb_lookup(xp_hbm, xq_hbm, tpair_hbm, out_hbm, pidx_v, poff_v,
                gbuf, sbuf, gsem0, gsem1, ssem0, ssem1):
    gsems = (gsem0, gsem1)
    ssems = (ssem0, ssem1)
    wid = lax.axis_index("s") * NC + lax.axis_index("c")
    base = wid * B_PER_W

    pltpu.sync_copy(xp_hbm.at[wid], pidx_v)
    pltpu.sync_copy(xq_hbm.at[wid], poff_v)

    def gather_copy(g, slot):
        return pltpu.make_async_copy(
            tpair_hbm.at[pidx_v.at[g]], gbuf.at[slot], gsems[slot])

    def scatter_copy(g, slot):
        return pltpu.make_async_copy(
            sbuf.at[slot], out_hbm.at[pl.ds(base + g * CHUNK, CHUNK)],
            ssems[slot])

    def scale_chunk(g, slot):
        def grp(k, carry):
            offs = poff_v[g, pl.ds(k * LANES, LANES)]
            for l in range(LANES):
                off = offs[l]
                r = k * LANES + l
                for c in range(D_MODEL // LANES):
                    sbuf[slot, r, pl.ds(c * LANES, LANES)] = (
                        gbuf[slot, r, pl.ds(off + c * LANES, LANES)] * SCALE)
            return carry
        lax.fori_loop(0, CHUNK // LANES, grp, 0)

    for b in range(NBUF):
        gather_copy(b, b).start()

    for b in range(NBUF):
        gather_copy(b, b).wait()
        scale_chunk(b, b)
        gather_copy(b + NBUF, b).start()
        scatter_copy(b, b).start()

    def outer(j, carry):
        for b in range(NBUF):
            g = j * NBUF + b
            gather_copy(g, b).wait()
            scatter_copy(g - NBUF, b).wait()
            scale_chunk(g, b)
            gather_copy(g + NBUF, b).start()
            scatter_copy(g, b).start()
        return carry
    lax.fori_loop(1, N_CHUNKS // NBUF - 1, outer, 0)

    for b in range(NBUF):
        g = N_CHUNKS - NBUF + b
        gather_copy(g, b).wait()
        scatter_copy(g - NBUF, b).wait()
        scale_chunk(g, b)
        scatter_copy(g, b).start()

    for b in range(NBUF):
        scatter_copy(N_CHUNKS - NBUF + b, b).wait()


_TBLK = 16384
_TSH = 14


def _transpose_body(xt_ref, out_ref):
    lo = xt_ref[:, : _TBLK // 2].T
    hi = xt_ref[:, _TBLK // 2 :].T
    out_ref[...] = lax.concatenate([lo, hi], 1)


_NBLK = -(-1000000 // _TBLK)

_transpose_table = pl.pallas_call(
    _transpose_body,
    grid=(_NBLK,),
    in_specs=[pl.BlockSpec((D_MODEL, _TBLK), lambda i: (0, i))],
    out_specs=pl.BlockSpec((_TBLK // 2, 2 * D_MODEL), lambda i: (i, 0)),
    out_shape=jax.ShapeDtypeStruct((_NBLK * _TBLK // 2, 2 * D_MODEL),
                                   jnp.float32),
)


def kernel(x, emb_table):
    xi = x.astype(jnp.int32)
    xp = (((xi >> _TSH) << (_TSH - 1))
          | (xi & (_TBLK // 2 - 1))).reshape(NW, N_CHUNKS, CHUNK)
    xq = (((xi >> (_TSH - 1)) & 1) * D_MODEL).reshape(NW, N_CHUNKS, CHUNK)
    tpair = _transpose_table(emb_table.T)
    out = _emb_lookup(xp, xq, tpair)
    return out.reshape(x.shape[0], x.shape[1], D_MODEL)


# speedup vs baseline: 1.5862x; 1.0199x over previous
import functools

import jax
import jax.numpy as jnp
from jax import lax
from jax.experimental import pallas as pl
from jax.experimental.pallas import tpu as pltpu
from jax.experimental.pallas import tpu_sc as plsc

D_MODEL = 64
SCALE = 8.0
LANES = 16

NC = 2
NS = 16
NW = NC * NS

B_TOTAL = 1024 * 200
B_PER_W = B_TOTAL // NW
CHUNK = 128
N_CHUNKS = B_PER_W // CHUNK
NBUF = 3

_mesh = plsc.VectorSubcoreMesh(core_axis_name="c", subcore_axis_name="s")


@functools.partial(
    pl.kernel,
    mesh=_mesh,
    compiler_params=pltpu.CompilerParams(use_tc_tiling_on_sc=True),
    out_type=jax.ShapeDtypeStruct((B_TOTAL, D_MODEL), jnp.float32),
    scratch_types=[
        pltpu.VMEM((N_CHUNKS, CHUNK), jnp.int32),
        pltpu.VMEM((N_CHUNKS, CHUNK), jnp.int32),
        pltpu.VMEM((NBUF, CHUNK, 2 * D_MODEL), jnp.float32),
        pltpu.VMEM((NBUF, CHUNK, D_MODEL), jnp.float32),
        pltpu.SemaphoreType.DMA,
        pltpu.SemaphoreType.DMA,
        pltpu.SemaphoreType.DMA,
        pltpu.SemaphoreType.DMA,
        pltpu.SemaphoreType.DMA,
        pltpu.SemaphoreType.DMA,
    ],
)
def _emb_lookup(xp_hbm, xq_hbm, tpair_hbm, out_hbm, pidx_v, poff_v,
                gbuf, sbuf, gsem0, gsem1, gsem2,
                ssem0, ssem1, ssem2):
    gsems = (gsem0, gsem1, gsem2)
    ssems = (ssem0, ssem1, ssem2)
    wid = lax.axis_index("s") * NC + lax.axis_index("c")
    base = wid * B_PER_W

    pltpu.sync_copy(xp_hbm.at[wid], pidx_v)
    pltpu.sync_copy(xq_hbm.at[wid], poff_v)

    def gather_copy(g, slot):
        return pltpu.make_async_copy(
            tpair_hbm.at[pidx_v.at[g]], gbuf.at[slot], gsems[slot])

    def scatter_copy(g, slot):
        return pltpu.make_async_copy(
            sbuf.at[slot], out_hbm.at[pl.ds(base + g * CHUNK, CHUNK)],
            ssems[slot])

    def scale_chunk(g, slot):
        def grp(k, carry):
            offs = poff_v[g, pl.ds(k * LANES, LANES)]
            for l in range(LANES):
                off = offs[l]
                r = k * LANES + l
                for c in range(D_MODEL // LANES):
                    sbuf[slot, r, pl.ds(c * LANES, LANES)] = (
                        gbuf[slot, r, pl.ds(off + c * LANES, LANES)] * SCALE)
            return carry
        lax.fori_loop(0, CHUNK // LANES, grp, 0)

    def process(g, slot, wait_scatter, start_next):
        gather_copy(g, slot).wait()
        if wait_scatter:
            scatter_copy(g - NBUF, slot).wait()
        scale_chunk(g, slot)
        if start_next:
            gather_copy(g + NBUF, slot).start()
        scatter_copy(g, slot).start()

    for b in range(NBUF):
        gather_copy(b, b).start()

    for b in range(NBUF):
        process(b, b, wait_scatter=False, start_next=True)

    _steady_hi = (N_CHUNKS - 2) // NBUF
    def outer(j, carry):
        for b in range(NBUF):
            process(j * NBUF + b, b, wait_scatter=True, start_next=True)
        return carry
    lax.fori_loop(1, _steady_hi - 1, outer, 0)

    for g in range((_steady_hi - 1) * NBUF, N_CHUNKS):
        process(g, g % NBUF, wait_scatter=True,
                start_next=(g + NBUF < N_CHUNKS))

    for g in range(N_CHUNKS - NBUF, N_CHUNKS):
        scatter_copy(g, g % NBUF).wait()


_TBLK = 32768
_TSH = 15


def _transpose_body(xt_ref, out_ref):
    lo = xt_ref[:, : _TBLK // 2].T
    hi = xt_ref[:, _TBLK // 2 :].T
    out_ref[...] = lax.concatenate([lo, hi], 1)


_NBLK = -(-1000000 // _TBLK)

_transpose_table = pl.pallas_call(
    _transpose_body,
    grid=(_NBLK,),
    in_specs=[pl.BlockSpec((D_MODEL, _TBLK), lambda i: (0, i))],
    out_specs=pl.BlockSpec((_TBLK // 2, 2 * D_MODEL), lambda i: (i, 0)),
    out_shape=jax.ShapeDtypeStruct((_NBLK * _TBLK // 2, 2 * D_MODEL),
                                   jnp.float32),
)


def kernel(x, emb_table):
    xi = x.astype(jnp.int32)
    xp = (((xi >> _TSH) << (_TSH - 1))
          | (xi & (_TBLK // 2 - 1))).reshape(NW, N_CHUNKS, CHUNK)
    xq = (((xi >> (_TSH - 1)) & 1) * D_MODEL).reshape(NW, N_CHUNKS, CHUNK)
    tpair = _transpose_table(emb_table.T)
    out = _emb_lookup(xp, xq, tpair)
    return out.reshape(x.shape[0], x.shape[1], D_MODEL)


# speedup vs baseline: 1.8074x; 1.1395x over previous
import functools

import jax
import jax.numpy as jnp
from jax import lax
from jax.experimental import pallas as pl
from jax.experimental.pallas import tpu as pltpu
from jax.experimental.pallas import tpu_sc as plsc

D_MODEL = 64
SCALE = 8.0
LANES = 16

NC = 2
NS = 16
NW = NC * NS

B_TOTAL = 1024 * 200
B_PER_W = B_TOTAL // NW
CHUNK = 128
N_CHUNKS = B_PER_W // CHUNK
NBUF = 3

_mesh = plsc.VectorSubcoreMesh(core_axis_name="c", subcore_axis_name="s")


@functools.partial(
    pl.kernel,
    mesh=_mesh,
    compiler_params=pltpu.CompilerParams(use_tc_tiling_on_sc=True),
    out_type=jax.ShapeDtypeStruct((B_TOTAL, D_MODEL), jnp.float32),
    scratch_types=[
        pltpu.VMEM((N_CHUNKS, CHUNK), jnp.int32),
        pltpu.VMEM((N_CHUNKS, CHUNK), jnp.int32),
        pltpu.VMEM((NBUF, CHUNK, 2 * D_MODEL), jnp.float32),
        pltpu.VMEM((NBUF, CHUNK, D_MODEL), jnp.float32),
        pltpu.SemaphoreType.DMA,
        pltpu.SemaphoreType.DMA,
        pltpu.SemaphoreType.DMA,
        pltpu.SemaphoreType.DMA,
        pltpu.SemaphoreType.DMA,
        pltpu.SemaphoreType.DMA,
    ],
)
def _emb_lookup(xp_hbm, xq_hbm, tpair_hbm, out_hbm, pidx_v, poff_v,
                gbuf, sbuf, gsem0, gsem1, gsem2,
                ssem0, ssem1, ssem2):
    gsems = (gsem0, gsem1, gsem2)
    ssems = (ssem0, ssem1, ssem2)
    wid = lax.axis_index("s") * NC + lax.axis_index("c")
    base = wid * B_PER_W

    pltpu.sync_copy(xp_hbm.at[wid], pidx_v)
    pltpu.sync_copy(xq_hbm.at[wid], poff_v)

    def gather_copy(g, slot):
        return pltpu.make_async_copy(
            tpair_hbm.at[pidx_v.at[g]], gbuf.at[slot], gsems[slot])

    def scatter_copy(g, slot):
        return pltpu.make_async_copy(
            sbuf.at[slot], out_hbm.at[pl.ds(base + g * CHUNK, CHUNK)],
            ssems[slot])

    def scale_chunk(g, slot):
        def grp(k, carry):
            offs = poff_v[g, pl.ds(k * LANES, LANES)]
            for l in range(LANES):
                off = offs[l]
                r = k * LANES + l
                for c in range(D_MODEL // LANES):
                    sbuf[slot, r, pl.ds(c * LANES, LANES)] = (
                        gbuf[slot, r, pl.ds(off + c * LANES, LANES)] * SCALE)
            return carry
        lax.fori_loop(0, CHUNK // LANES, grp, 0)

    def process(g, slot, wait_scatter, start_next):
        gather_copy(g, slot).wait()
        if wait_scatter:
            scatter_copy(g - NBUF, slot).wait()
        scale_chunk(g, slot)
        if start_next:
            gather_copy(g + NBUF, slot).start()
        scatter_copy(g, slot).start()

    for b in range(NBUF):
        gather_copy(b, b).start()

    for b in range(NBUF):
        process(b, b, wait_scatter=False, start_next=True)

    _steady_hi = (N_CHUNKS - 2) // NBUF
    def outer(j, carry):
        for b in range(NBUF):
            process(j * NBUF + b, b, wait_scatter=True, start_next=True)
        return carry
    lax.fori_loop(1, _steady_hi - 1, outer, 0)

    for g in range((_steady_hi - 1) * NBUF, N_CHUNKS):
        process(g, g % NBUF, wait_scatter=True,
                start_next=(g + NBUF < N_CHUNKS))

    for g in range(N_CHUNKS - NBUF, N_CHUNKS):
        scatter_copy(g, g % NBUF).wait()


_TBLK = 32768
_TSH = 15


def _transpose_body(xt_ref, out_ref):
    stacked = lax.concatenate(
        [xt_ref[:, : _TBLK // 2], xt_ref[:, _TBLK // 2 :]], 0)
    out_ref[...] = stacked.T


_NBLK = -(-1000000 // _TBLK)

_transpose_table = pl.pallas_call(
    _transpose_body,
    grid=(_NBLK,),
    in_specs=[pl.BlockSpec((D_MODEL, _TBLK), lambda i: (0, i))],
    out_specs=pl.BlockSpec((_TBLK // 2, 2 * D_MODEL), lambda i: (i, 0)),
    out_shape=jax.ShapeDtypeStruct((_NBLK * _TBLK // 2, 2 * D_MODEL),
                                   jnp.float32),
)


def kernel(x, emb_table):
    xi = x.astype(jnp.int32)
    xp = (((xi >> _TSH) << (_TSH - 1))
          | (xi & (_TBLK // 2 - 1))).reshape(NW, N_CHUNKS, CHUNK)
    xq = (((xi >> (_TSH - 1)) & 1) * D_MODEL).reshape(NW, N_CHUNKS, CHUNK)
    tpair = _transpose_table(emb_table.T)
    out = _emb_lookup(xp, xq, tpair)
    return out.reshape(x.shape[0], x.shape[1], D_MODEL)
